# Initial kernel scaffold; baseline (speedup 1.0000x reference)
import functools
import jax
import jax.numpy as jnp
from jax import lax
from jax.experimental import pallas as pl
from jax.experimental.pallas import tpu as pltpu
from jax.experimental.pallas import tpu_sc as plsc

N = 10000
E = 320000
D = 128
H = 256
DE = 16
OUT = 2

NC = 2
NS = 16
NW = NC * NS
NR = 320
NPAD = NW * NR
CE = 128
TRASH = NR
EPAD1 = E + CE + 8

EW = E // NW
CM = 128
KCH = (EW + CM - 1) // CM
EPAD2 = (NW - 1) * EW + KCH * CM
EPAD2 = ((EPAD2 + 127) // 128) * 128
OUTP = 8


def _extract_i32(vref, pos, nvec):
    s = jnp.int32(-1)
    for i in range(nvec):
        v = vref[pl.ds(i * 16, 16)]
        idx = lax.iota(jnp.int32, 16) + jnp.int32(i * 16)
        cand = jnp.max(jnp.where(idx == pos, v, jnp.int32(-1)))
        s = jnp.maximum(s, cand)
    return s


def _make_sc_agg(width, gather):
    mesh = plsc.VectorSubcoreMesh(core_axis_name="c", subcore_axis_name="s")
    scratch = [
        pltpu.VMEM((48,), jnp.int32),
        pltpu.VMEM((CE,), jnp.int32),
        pltpu.VMEM((CE,), jnp.int32),
        pltpu.VMEM((1, CE), jnp.int32),
        pltpu.VMEM((CE, width), jnp.float32),
        pltpu.VMEM((NR + 8, width), jnp.float32),
        pltpu.SemaphoreType.DMA,
    ]

    @functools.partial(
        pl.kernel,
        out_type=jax.ShapeDtypeStruct((NPAD, width), jnp.float32),
        mesh=mesh,
        scratch_types=scratch,
    )
    def k(table, srcs, dsts, estart, out, ebuf, sbuf, dbuf, lbuf, G, acc, sem):
        c = lax.axis_index("c")
        s = lax.axis_index("s")
        w = c * NS + s
        n0 = w * NR
        pltpu.sync_copy(estart, ebuf)
        e0 = _extract_i32(ebuf, w, 3)
        e1 = _extract_i32(ebuf, w + 1, 3)
        e0a = (e0 // 8) * 8
        nch = (e1 - e0a + (CE - 1)) // CE
        pltpu.sync_copy(table.at[pl.ds(n0, NR)], acc.at[pl.ds(0, NR)])
        if not gather:
            ones = jnp.full((16,), 1.0, jnp.float32)
            for rr in range(CE):
                for j in range(width // 16):
                    G[rr, pl.ds(j * 16, 16)] = ones

        def chunk(kk, carry):
            off = e0a + kk * CE
            pltpu.sync_copy(dsts.at[pl.ds(off, CE)], dbuf)
            if gather:
                pltpu.sync_copy(srcs.at[pl.ds(off, CE)], sbuf)
                pltpu.async_copy(table.at[sbuf], G, sem).wait()
            for i in range(CE // 16):
                dv = dbuf[pl.ds(i * 16, 16)]
                pos = lax.iota(jnp.int32, 16) + (off + jnp.int32(i * 16))
                valid = (pos >= e0) & (pos < e1)
                ld = jnp.where(valid, dv - n0, jnp.int32(TRASH))
                lbuf[0, pl.ds(i * 16, 16)] = ld
            pltpu.sync_copy(G, acc.at[lbuf.at[0]], add=True)
            return carry

        lax.fori_loop(0, nch, chunk, jnp.int32(0))
        pltpu.sync_copy(acc.at[pl.ds(0, NR)], out.at[pl.ds(n0, NR)])

    return k


_sc_deg = _make_sc_agg(16, gather=False)
_sc_agg_d = _make_sc_agg(D, gather=True)
_sc_agg_h = _make_sc_agg(H, gather=True)


def _make_sc_mlp_gather():
    mesh = plsc.VectorSubcoreMesh(core_axis_name="c", subcore_axis_name="s")
    scratch = [
        pltpu.VMEM((CM,), jnp.int32),
        pltpu.VMEM((CM,), jnp.int32),
        pltpu.VMEM((1, CM), jnp.int32),
        pltpu.VMEM((CM, H), jnp.float32),
        pltpu.VMEM((CM, H), jnp.float32),
        pltpu.SemaphoreType.DMA,
        pltpu.SemaphoreType.DMA,
    ]

    @functools.partial(
        pl.kernel,
        out_type=jax.ShapeDtypeStruct((EPAD2, H), jnp.float32),
        mesh=mesh,
        scratch_types=scratch,
    )
    def k(A, B, srcu, dstu, out, sbuf, dbuf, ibuf, GA, GB, sem, sem2):
        c = lax.axis_index("c")
        s = lax.axis_index("s")
        w = c * NS + s
        base = w * EW
        for i in range(CM // 16):
            ibuf[0, pl.ds(i * 16, 16)] = lax.iota(jnp.int32, 16) + jnp.int32(i * 16)

        def chunk(kk, carry):
            off = base + kk * CM
            pltpu.sync_copy(srcu.at[pl.ds(off, CM)], sbuf)
            pltpu.sync_copy(dstu.at[pl.ds(off, CM)], dbuf)
            pltpu.async_copy(A.at[sbuf], GA, sem).wait()
            pltpu.async_copy(B.at[dbuf], GB, sem2).wait()
            pltpu.sync_copy(GB, GA.at[ibuf.at[0]], add=True)
            pltpu.sync_copy(GA, out.at[pl.ds(off, CM)])
            return carry

        lax.fori_loop(0, KCH, chunk, jnp.int32(0))

    return k


_sc_mlp_gather = _make_sc_mlp_gather()


def _tc0_body(x_ref, degc_ref, y_ref):
    deg = degc_ref[0:N, 0] + 1.0
    dis = 1.0 / jnp.sqrt(deg)
    xx = jnp.nan_to_num(x_ref[...])
    y_ref[0:N, :] = xx * dis[:, None]
    y_ref[N:NPAD, :] = jnp.zeros((NPAD - N, D), jnp.float32)


def _tc0(x, degc):
    return pl.pallas_call(
        _tc0_body,
        out_shape=jax.ShapeDtypeStruct((NPAD, D), jnp.float32),
    )(x, degc)


def _make_tc_layer(din, has_prev, emit_ab):
    def body(*refs):
        i = 0
        agg_ref = refs[i]; i += 1
        degc_ref = refs[i]; i += 1
        prev_ref = None
        if has_prev:
            prev_ref = refs[i]; i += 1
        W_ref = refs[i]; i += 1
        b_ref = refs[i]; i += 1
        g_ref = refs[i]; i += 1
        be_ref = refs[i]; i += 1
        if emit_ab:
            cwa_ref = refs[i]; i += 1
            cwb_ref = refs[i]; i += 1
            a_out, b_out = refs[i], refs[i + 1]
        else:
            x_out, y_out = refs[i], refs[i + 1]

        deg = degc_ref[0:N, 0] + 1.0
        dis = 1.0 / jnp.sqrt(deg)
        z = agg_ref[0:N, :] * dis[:, None]
        gcn = jnp.dot(z, W_ref[...], preferred_element_type=jnp.float32) + b_ref[...]
        mu = jnp.mean(gcn, axis=0)
        xc = gcn - mu
        var = jnp.mean(xc * xc, axis=0)
        v = g_ref[...] * xc / jnp.sqrt(var + 1e-5) + be_ref[...]
        v = jnp.maximum(v, 0.0)
        if has_prev:
            v = v + prev_ref[...]
        if emit_ab:
            a_out[...] = jnp.dot(v, cwa_ref[...], preferred_element_type=jnp.float32)
            b_out[...] = jnp.dot(v, cwb_ref[...], preferred_element_type=jnp.float32)
        else:
            x_out[...] = v
            y_out[0:N, :] = v * dis[:, None]
            y_out[N:NPAD, :] = jnp.zeros((NPAD - N, H), jnp.float32)

    if emit_ab:
        out_shape = (jax.ShapeDtypeStruct((N, H), jnp.float32),
                     jax.ShapeDtypeStruct((N, H), jnp.float32))
    else:
        out_shape = (jax.ShapeDtypeStruct((N, H), jnp.float32),
                     jax.ShapeDtypeStruct((NPAD, H), jnp.float32))

    def run(*args):
        return pl.pallas_call(body, out_shape=out_shape)(*args)

    return run


_tc_layer1 = _make_tc_layer(D, has_prev=False, emit_ab=False)
_tc_layer23 = _make_tc_layer(H, has_prev=True, emit_ab=False)
_tc_layer4 = _make_tc_layer(H, has_prev=True, emit_ab=True)

BB = 512


def _tc_mlp_body(g1_ref, ea_ref, cw1e_ref, cb1_ref, cw2_ref, cb2_ref,
                 cw3_ref, cb3_ref, cw4_ref, cb4_ref, out_ref):
    ea = jnp.nan_to_num(ea_ref[...])
    f1 = g1_ref[...] + jnp.dot(ea, cw1e_ref[...], preferred_element_type=jnp.float32)
    f1 = jnp.maximum(f1 + cb1_ref[...], 0.0)
    h2 = jnp.maximum(jnp.dot(f1, cw2_ref[...], preferred_element_type=jnp.float32)
                     + cb2_ref[...], 0.0)
    h3 = jnp.maximum(jnp.dot(h2, cw3_ref[...], preferred_element_type=jnp.float32)
                     + cb3_ref[...], 0.0)
    out_ref[...] = jnp.dot(h3, cw4_ref[...], preferred_element_type=jnp.float32) + cb4_ref[...]


def _tc_mlp(g1, ea, cw1e, cb1, cw2, cb2, cw3, cb3, cw4p, cb4p):
    nblk = E // BB
    full = lambda i: (0, 0)
    return pl.pallas_call(
        _tc_mlp_body,
        grid=(nblk,),
        in_specs=[
            pl.BlockSpec((BB, H), lambda i: (i, 0)),
            pl.BlockSpec((BB, DE), lambda i: (i, 0)),
            pl.BlockSpec((DE, H), full),
            pl.BlockSpec((1, H), full),
            pl.BlockSpec((H, H // 2), full),
            pl.BlockSpec((1, H // 2), full),
            pl.BlockSpec((H // 2, H // 4), full),
            pl.BlockSpec((1, H // 4), full),
            pl.BlockSpec((H // 4, OUTP), full),
            pl.BlockSpec((1, OUTP), full),
        ],
        out_specs=pl.BlockSpec((BB, OUTP), lambda i: (i, 0)),
        out_shape=jax.ShapeDtypeStruct((E, OUTP), jnp.float32),
    )(g1, ea, cw1e, cb1, cw2, cb2, cw3, cb3, cw4p, cb4p)


def kernel(x, edge_index, edge_attr, W1, b1, g1, be1, W2, b2, g2, be2,
           W3, b3, g3, be3, W4, b4, g4, be4,
           CW1, Cb1, CW2, Cb2, CW3, Cb3, CW4, Cb4):
    src = edge_index[0].astype(jnp.int32)
    dst = edge_index[1].astype(jnp.int32)

    perm = jnp.argsort(dst)
    dsts = dst[perm]
    srcs = src[perm]
    srcs_p = jnp.concatenate([srcs, jnp.zeros((EPAD1 - E,), jnp.int32)])
    dsts_p = jnp.concatenate([dsts, jnp.zeros((EPAD1 - E,), jnp.int32)])
    cuts = jnp.arange(NW + 1, dtype=jnp.int32) * NR
    estart = jnp.searchsorted(dsts, cuts).astype(jnp.int32)
    estart_p = jnp.concatenate([estart, jnp.full((48 - (NW + 1),), E, jnp.int32)])

    zeros16 = jnp.zeros((NPAD, 16), jnp.float32)
    degc = _sc_deg(zeros16, srcs_p, dsts_p, estart_p)

    y1 = _tc0(x, degc)
    r = lambda a: a.reshape(1, -1)

    agg1 = _sc_agg_d(y1, srcs_p, dsts_p, estart_p)
    x1, y2 = _tc_layer1(agg1, degc, W1, r(b1), r(g1), r(be1))

    agg2 = _sc_agg_h(y2, srcs_p, dsts_p, estart_p)
    x2, y3 = _tc_layer23(agg2, degc, x1, W2, r(b2), r(g2), r(be2))

    agg3 = _sc_agg_h(y3, srcs_p, dsts_p, estart_p)
    x3, y4 = _tc_layer23(agg3, degc, x2, W3, r(b3), r(g3), r(be3))

    agg4 = _sc_agg_h(y4, srcs_p, dsts_p, estart_p)
    A, B = _tc_layer4(agg4, degc, x3, W4, r(b4), r(g4), r(be4),
                      CW1[:H], CW1[H:2 * H])

    srcu_p = jnp.concatenate([src, jnp.zeros((EPAD2 - E,), jnp.int32)])
    dstu_p = jnp.concatenate([dst, jnp.zeros((EPAD2 - E,), jnp.int32)])
    g1e = _sc_mlp_gather(A, B, srcu_p, dstu_p)

    cw4p = jnp.pad(CW4, ((0, 0), (0, OUTP - OUT)))
    cb4p = jnp.pad(Cb4, (0, OUTP - OUT))
    outp = _tc_mlp(g1e, edge_attr, CW1[2 * H:], r(Cb1), CW2, r(Cb2),
                   CW3, r(Cb3), cw4p, r(cb4p))
    return outp[:, :OUT]


# speedup vs baseline: 5.7962x; 5.7962x over previous
import functools
import jax
import jax.numpy as jnp
from jax import lax
from jax.experimental import pallas as pl
from jax.experimental.pallas import tpu as pltpu
from jax.experimental.pallas import tpu_sc as plsc

N = 10000
E = 320000
D = 128
H = 256
DE = 16
OUT = 2

NC = 2
NS = 16
NW = NC * NS
NR = 320
NPAD = NW * NR
CE = 128
TRASH = NR
EPAD1 = E + CE + 8

EW = E // NW
CM = 128
KCH = (EW + CM - 1) // CM
EPAD2 = (NW - 1) * EW + KCH * CM
EPAD2 = ((EPAD2 + 127) // 128) * 128
OUTP = 8


@functools.lru_cache(maxsize=None)
def _make_sc_agg(width, ngroups, gather):
    mesh = plsc.VectorSubcoreMesh(core_axis_name="c", subcore_axis_name="s")
    scratch = [
        pltpu.VMEM((16,), jnp.int32),
        pltpu.VMEM((CE,), jnp.int32),
        pltpu.VMEM((CE,), jnp.int32),
        pltpu.VMEM((1, CE), jnp.int32),
        pltpu.VMEM((CE, width), jnp.float32),
        pltpu.SemaphoreType.DMA,
    ] + [
        pltpu.VMEM_SHARED((NS * (NR + 8), width), jnp.float32)
        for _ in range(ngroups)
    ]

    @functools.partial(
        pl.kernel,
        out_type=tuple(jax.ShapeDtypeStruct((NPAD, width), jnp.float32)
                       for _ in range(ngroups)),
        mesh=mesh,
        scratch_types=scratch,
    )
    def k(*refs):
        tables = refs[:ngroups]
        srcs, dsts, epairs = refs[ngroups:ngroups + 3]
        outs = refs[ngroups + 3:2 * ngroups + 3]
        ebuf, sbuf, dbuf, lbuf, G, sem = refs[2 * ngroups + 3:2 * ngroups + 9]
        accs = refs[2 * ngroups + 9:]

        c = lax.axis_index("c")
        s = lax.axis_index("s")
        w = c * NS + s
        n0 = w * NR
        pltpu.sync_copy(epairs.at[w], ebuf)
        ev = ebuf[...]
        e0 = ev[0]
        e1 = ev[1]
        e0a = (e0 // 8) * 8
        nch = (e1 - e0a + (CE - 1)) // CE
        a0 = s * (NR + 8)
        for g in range(ngroups):
            pltpu.sync_copy(tables[g].at[pl.ds(n0, NR)], accs[g].at[pl.ds(a0, NR)])
        if not gather:
            ones = jnp.full((16,), 1.0, jnp.float32)
            for rr in range(CE):
                for j in range(width // 16):
                    G[rr, pl.ds(j * 16, 16)] = ones

        def chunk(kk, carry):
            off = e0a + kk * CE
            pltpu.sync_copy(dsts.at[pl.ds(off, CE)], dbuf)
            if gather:
                pltpu.sync_copy(srcs.at[pl.ds(off, CE)], sbuf)
            for i in range(CE // 16):
                dv = dbuf[pl.ds(i * 16, 16)]
                pos = lax.iota(jnp.int32, 16) + (off + jnp.int32(i * 16))
                valid = (pos >= e0) & (pos < e1)
                ld = jnp.where(valid, (dv - n0) + a0, jnp.int32(TRASH) + a0)
                lbuf[0, pl.ds(i * 16, 16)] = ld
            for g in range(ngroups):
                if gather:
                    pltpu.async_copy(tables[g].at[sbuf], G, sem).wait()
                pltpu.sync_copy(G, accs[g].at[lbuf.at[0]], add=True)
            return carry

        lax.fori_loop(0, nch, chunk, jnp.int32(0))
        for g in range(ngroups):
            pltpu.sync_copy(accs[g].at[pl.ds(a0, NR)], outs[g].at[pl.ds(n0, NR)])

    return k


def _sc_deg(*a):
    return _make_sc_agg(16, 1, gather=False)(*a)[0]


def _sc_agg_d(*a):
    return _make_sc_agg(D, 1, gather=True)(*a)[0]


def _sc_agg_h(*a):
    return _make_sc_agg(128, 2, gather=True)(*a)


@functools.lru_cache(maxsize=None)
def _make_sc_mlp_gather():
    mesh = plsc.VectorSubcoreMesh(core_axis_name="c", subcore_axis_name="s")
    scratch = [
        pltpu.VMEM((CM,), jnp.int32),
        pltpu.VMEM((CM,), jnp.int32),
        pltpu.VMEM((CM, H), jnp.float32),
        pltpu.VMEM((CM, H), jnp.float32),
        pltpu.SemaphoreType.DMA,
        pltpu.SemaphoreType.DMA,
    ]

    @functools.partial(
        pl.kernel,
        out_type=(jax.ShapeDtypeStruct((EPAD2, H), jnp.float32),
                  jax.ShapeDtypeStruct((EPAD2, H), jnp.float32)),
        mesh=mesh,
        scratch_types=scratch,
    )
    def k(A, B, srcu, dstu, outa, outb, sbuf, dbuf, GA, GB, sem, sem2):
        c = lax.axis_index("c")
        s = lax.axis_index("s")
        w = c * NS + s
        base = w * EW

        def chunk(kk, carry):
            off = base + kk * CM
            pltpu.sync_copy(srcu.at[pl.ds(off, CM)], sbuf)
            pltpu.sync_copy(dstu.at[pl.ds(off, CM)], dbuf)
            pltpu.async_copy(A.at[sbuf], GA, sem).wait()
            pltpu.async_copy(B.at[dbuf], GB, sem2).wait()
            pltpu.sync_copy(GA, outa.at[pl.ds(off, CM)])
            pltpu.sync_copy(GB, outb.at[pl.ds(off, CM)])
            return carry

        lax.fori_loop(0, KCH, chunk, jnp.int32(0))

    return k


def _sc_mlp_gather(*a):
    return _make_sc_mlp_gather()(*a)


def _tc0_body(x_ref, degc_ref, y_ref):
    deg = degc_ref[0:N, 0] + 1.0
    dis = 1.0 / jnp.sqrt(deg)
    xx = jnp.nan_to_num(x_ref[...])
    y_ref[0:N, :] = xx * dis[:, None]
    y_ref[N:NPAD, :] = jnp.zeros((NPAD - N, D), jnp.float32)


def _tc0(x, degc):
    return pl.pallas_call(
        _tc0_body,
        out_shape=jax.ShapeDtypeStruct((NPAD, D), jnp.float32),
    )(x, degc)


def _make_tc_layer(nagg, has_prev, last):
    def body(*refs):
        i = 0
        agg_refs = refs[:nagg]; i += nagg
        prevs = ()
        if has_prev:
            prevs = refs[i], refs[i + 1]; i += 2
        degc_ref = refs[i]; i += 1
        W_ref = refs[i]; i += 1
        b_ref = refs[i]; i += 1
        g_ref = refs[i]; i += 1
        be_ref = refs[i]; i += 1
        oa, ob = refs[i], refs[i + 1]

        deg = degc_ref[0:N, 0] + 1.0
        dis = 1.0 / jnp.sqrt(deg)
        if nagg > 1:
            gcn = (jnp.dot(agg_refs[0][0:N, :] * dis[:, None], W_ref[0:128, :],
                           preferred_element_type=jnp.float32)
                   + jnp.dot(agg_refs[1][0:N, :] * dis[:, None], W_ref[128:H, :],
                             preferred_element_type=jnp.float32))
        else:
            gcn = jnp.dot(agg_refs[0][0:N, :] * dis[:, None], W_ref[...],
                          preferred_element_type=jnp.float32)
        gcn = gcn + b_ref[...]
        mu = jnp.mean(gcn, axis=0)
        xc = gcn - mu
        var = jnp.mean(xc * xc, axis=0)
        v = g_ref[...] * xc / jnp.sqrt(var + 1e-5) + be_ref[...]
        v = jnp.maximum(v, 0.0)
        if last:
            rd = jnp.sqrt(deg)
            pa = prevs[0][0:N, :] * rd[:, None] + v[:, 0:128]
            pb = prevs[1][0:N, :] * rd[:, None] + v[:, 128:H]
            oa[...] = pa
            ob[...] = pb
        else:
            y = v * dis[:, None]
            ya = y[:, 0:128]
            yb = y[:, 128:H]
            if has_prev:
                ya = ya + prevs[0][0:N, :]
                yb = yb + prevs[1][0:N, :]
            oa[0:N, :] = ya
            oa[N:NPAD, :] = jnp.zeros((NPAD - N, 128), jnp.float32)
            ob[0:N, :] = yb
            ob[N:NPAD, :] = jnp.zeros((NPAD - N, 128), jnp.float32)

    rows = N if last else NPAD
    out_shape = (jax.ShapeDtypeStruct((rows, 128), jnp.float32),
                 jax.ShapeDtypeStruct((rows, 128), jnp.float32))

    def run(*args):
        return pl.pallas_call(body, out_shape=out_shape)(*args)

    return run


_tc_layer1 = _make_tc_layer(1, has_prev=False, last=False)
_tc_layer23 = _make_tc_layer(2, has_prev=True, last=False)
_tc_layer4 = _make_tc_layer(2, has_prev=True, last=True)

BBN = 400


def _tc_ab_body(xa_ref, xb_ref, cwa_ref, cwb_ref, a_ref, b_ref):
    xa = xa_ref[...]
    xb = xb_ref[...]
    a_ref[...] = (jnp.dot(xa, cwa_ref[0:128, :], preferred_element_type=jnp.float32)
                  + jnp.dot(xb, cwa_ref[128:H, :], preferred_element_type=jnp.float32))
    b_ref[...] = (jnp.dot(xa, cwb_ref[0:128, :], preferred_element_type=jnp.float32)
                  + jnp.dot(xb, cwb_ref[128:H, :], preferred_element_type=jnp.float32))


def _tc_ab(xa, xb, cwa, cwb):
    full = lambda i: (0, 0)
    return pl.pallas_call(
        _tc_ab_body,
        grid=(N // BBN,),
        in_specs=[
            pl.BlockSpec((BBN, 128), lambda i: (i, 0)),
            pl.BlockSpec((BBN, 128), lambda i: (i, 0)),
            pl.BlockSpec((H, H), full),
            pl.BlockSpec((H, H), full),
        ],
        out_specs=[
            pl.BlockSpec((BBN, H), lambda i: (i, 0)),
            pl.BlockSpec((BBN, H), lambda i: (i, 0)),
        ],
        out_shape=[
            jax.ShapeDtypeStruct((N, H), jnp.float32),
            jax.ShapeDtypeStruct((N, H), jnp.float32),
        ],
    )(xa, xb, cwa, cwb)


BB = 512


def _tc_mlp_body(ga_ref, gb_ref, ea_ref, cw1e_ref, cb1_ref, cw2_ref, cb2_ref,
                 cw3_ref, cb3_ref, cw4_ref, cb4_ref, out_ref):
    ea = jnp.nan_to_num(ea_ref[...])
    f1 = ga_ref[...] + gb_ref[...] + jnp.dot(ea, cw1e_ref[...], preferred_element_type=jnp.float32)
    f1 = jnp.maximum(f1 + cb1_ref[...], 0.0)
    h2 = jnp.maximum(jnp.dot(f1, cw2_ref[...], preferred_element_type=jnp.float32)
                     + cb2_ref[...], 0.0)
    h3 = jnp.maximum(jnp.dot(h2, cw3_ref[...], preferred_element_type=jnp.float32)
                     + cb3_ref[...], 0.0)
    out_ref[...] = jnp.dot(h3, cw4_ref[...], preferred_element_type=jnp.float32) + cb4_ref[...]


def _tc_mlp(ga, gb, ea, cw1e, cb1, cw2, cb2, cw3, cb3, cw4p, cb4p):
    nblk = E // BB
    full = lambda i: (0, 0)
    return pl.pallas_call(
        _tc_mlp_body,
        grid=(nblk,),
        in_specs=[
            pl.BlockSpec((BB, H), lambda i: (i, 0)),
            pl.BlockSpec((BB, H), lambda i: (i, 0)),
            pl.BlockSpec((BB, DE), lambda i: (i, 0)),
            pl.BlockSpec((DE, H), full),
            pl.BlockSpec((1, H), full),
            pl.BlockSpec((H, H // 2), full),
            pl.BlockSpec((1, H // 2), full),
            pl.BlockSpec((H // 2, H // 4), full),
            pl.BlockSpec((1, H // 4), full),
            pl.BlockSpec((H // 4, OUTP), full),
            pl.BlockSpec((1, OUTP), full),
        ],
        out_specs=pl.BlockSpec((BB, OUTP), lambda i: (i, 0)),
        out_shape=jax.ShapeDtypeStruct((E, OUTP), jnp.float32),
    )(ga, gb, ea, cw1e, cb1, cw2, cb2, cw3, cb3, cw4p, cb4p)


def kernel(x, edge_index, edge_attr, W1, b1, g1, be1, W2, b2, g2, be2,
           W3, b3, g3, be3, W4, b4, g4, be4,
           CW1, Cb1, CW2, Cb2, CW3, Cb3, CW4, Cb4):
    src = edge_index[0].astype(jnp.int32)
    dst = edge_index[1].astype(jnp.int32)

    perm = jnp.argsort(dst)
    dsts = dst[perm]
    srcs = src[perm]
    srcs_p = jnp.concatenate([srcs, jnp.zeros((EPAD1 - E,), jnp.int32)])
    dsts_p = jnp.concatenate([dsts, jnp.zeros((EPAD1 - E,), jnp.int32)])
    cuts = jnp.arange(NW + 1, dtype=jnp.int32) * NR
    estart = jnp.searchsorted(dsts, cuts).astype(jnp.int32)
    epairs = jnp.concatenate(
        [estart[:-1, None], estart[1:, None],
         jnp.zeros((NW, 14), jnp.int32)], axis=1)

    zeros16 = jnp.zeros((NPAD, 16), jnp.float32)
    degc = _sc_deg(zeros16, srcs_p, dsts_p, epairs)

    y1 = _tc0(x, degc)
    r = lambda a: a.reshape(1, -1)

    agg1 = _sc_agg_d(y1, srcs_p, dsts_p, epairs)
    y2a, y2b = _tc_layer1(agg1, degc, W1, r(b1), r(g1), r(be1))

    agg2a, agg2b = _sc_agg_h(y2a, y2b, srcs_p, dsts_p, epairs)
    y3a, y3b = _tc_layer23(agg2a, agg2b, y2a, y2b, degc, W2, r(b2), r(g2), r(be2))

    agg3a, agg3b = _sc_agg_h(y3a, y3b, srcs_p, dsts_p, epairs)
    y4a, y4b = _tc_layer23(agg3a, agg3b, y3a, y3b, degc, W3, r(b3), r(g3), r(be3))

    agg4a, agg4b = _sc_agg_h(y4a, y4b, srcs_p, dsts_p, epairs)
    x4a, x4b = _tc_layer4(agg4a, agg4b, y4a, y4b, degc, W4, r(b4), r(g4), r(be4))
    A, B = _tc_ab(x4a, x4b, CW1[:H], CW1[H:2 * H])

    srcu_p = jnp.concatenate([src, jnp.zeros((EPAD2 - E,), jnp.int32)])
    dstu_p = jnp.concatenate([dst, jnp.zeros((EPAD2 - E,), jnp.int32)])
    ga, gb = _sc_mlp_gather(A, B, srcu_p, dstu_p)

    cw4p = jnp.pad(CW4, ((0, 0), (0, OUTP - OUT)))
    cb4p = jnp.pad(Cb4, (0, OUTP - OUT))
    outp = _tc_mlp(ga, gb, edge_attr, CW1[2 * H:], r(Cb1), CW2, r(Cb2),
                   CW3, r(Cb3), cw4p, r(cb4p))
    return outp[:, :OUT]
